# baseline (speedup 1.0000x reference)
import jax
import jax.numpy as jnp
from jax.experimental import pallas as pl
from jax.sharding import Mesh, NamedSharding, PartitionSpec as P

N_TOK = 8192
DIM = 4096
BM = 512
BCAST = 256


def _cast_kernel(w_ref, o_ref):
    o_ref[...] = w_ref[...].astype(jnp.bfloat16)


def _mm_kernel(x_ref, w_ref, b_ref, o_ref):
    xb = x_ref[...].astype(jnp.bfloat16)
    acc = jnp.dot(xb, w_ref[...], preferred_element_type=jnp.float32)
    o_ref[...] = acc + b_ref[...]


def _per_core(x, w_panel, b_panel):
    n_cols = w_panel.shape[1]
    wb = pl.pallas_call(
        _cast_kernel,
        grid=(DIM // BCAST,),
        in_specs=[pl.BlockSpec((BCAST, n_cols), lambda k: (k, 0))],
        out_specs=pl.BlockSpec((BCAST, n_cols), lambda k: (k, 0)),
        out_shape=jax.ShapeDtypeStruct((DIM, n_cols), jnp.bfloat16),
    )(w_panel)
    return pl.pallas_call(
        _mm_kernel,
        grid=(N_TOK // BM,),
        in_specs=[
            pl.BlockSpec((BM, DIM), lambda m: (m, 0)),
            pl.BlockSpec((DIM, n_cols), lambda m: (0, 0)),
            pl.BlockSpec((1, n_cols), lambda m: (0, 0)),
        ],
        out_specs=pl.BlockSpec((BM, n_cols), lambda m: (m, 0)),
        out_shape=jax.ShapeDtypeStruct((N_TOK, n_cols), jnp.float32),
    )(x, wb, b_panel)


def kernel(x, weight, bias):
    devs = jax.devices()
    b2 = bias.reshape(1, DIM)
    if len(devs) < 2:
        return _per_core(x, weight, b2)
    mesh = Mesh(devs[:2], ("d",))
    fn = jax.shard_map(
        _per_core,
        mesh=mesh,
        in_specs=(P(), P(None, "d"), P(None, "d")),
        out_specs=P(None, "d"),
        check_vma=False,
    )
    x = jax.lax.with_sharding_constraint(x, NamedSharding(mesh, P()))
    weight = jax.lax.with_sharding_constraint(
        weight, NamedSharding(mesh, P(None, "d"))
    )
    b2 = jax.lax.with_sharding_constraint(b2, NamedSharding(mesh, P(None, "d")))
    return fn(x, weight, b2)


# speedup vs baseline: 1.8935x; 1.8935x over previous
import jax
import jax.numpy as jnp
from jax.experimental import pallas as pl

N_TOK = 8192
DIM = 4096
BM = 512
BN = 1024
BCAST = 256


def _cast_kernel(w_ref, o_ref):
    o_ref[...] = w_ref[...].astype(jnp.bfloat16)


def _mm_kernel(x_ref, w_ref, b_ref, o_ref):
    xb = x_ref[...].astype(jnp.bfloat16)
    acc = jnp.dot(xb, w_ref[...], preferred_element_type=jnp.float32)
    o_ref[...] = acc + b_ref[...]


def kernel(x, weight, bias):
    wb = pl.pallas_call(
        _cast_kernel,
        grid=(DIM // BCAST,),
        in_specs=[pl.BlockSpec((BCAST, DIM), lambda k: (k, 0))],
        out_specs=pl.BlockSpec((BCAST, DIM), lambda k: (k, 0)),
        out_shape=jax.ShapeDtypeStruct((DIM, DIM), jnp.bfloat16),
    )(weight)
    b2 = bias.reshape(1, DIM)
    return pl.pallas_call(
        _mm_kernel,
        grid=(DIM // BN, N_TOK // BM),
        in_specs=[
            pl.BlockSpec((BM, DIM), lambda n, m: (m, 0)),
            pl.BlockSpec((DIM, BN), lambda n, m: (0, n)),
            pl.BlockSpec((1, BN), lambda n, m: (0, n)),
        ],
        out_specs=pl.BlockSpec((BM, BN), lambda n, m: (m, n)),
        out_shape=jax.ShapeDtypeStruct((N_TOK, DIM), jnp.float32),
    )(x, wb, b2)


# speedup vs baseline: 1.9602x; 1.0353x over previous
import jax
import jax.numpy as jnp
from jax.experimental import pallas as pl
from jax.experimental.pallas import tpu as pltpu

N_TOK = 8192
DIM = 4096
BM = 256


def _mm_kernel(x_ref, w_ref, b_ref, o_ref):
    xb = x_ref[...].astype(jnp.bfloat16)
    acc = jnp.dot(xb, w_ref[...], preferred_element_type=jnp.float32)
    o_ref[...] = acc + b_ref[...]


def kernel(x, weight, bias):
    wb = weight.astype(jnp.bfloat16)
    b2 = bias.reshape(1, DIM)
    return pl.pallas_call(
        _mm_kernel,
        grid=(N_TOK // BM,),
        in_specs=[
            pl.BlockSpec((BM, DIM), lambda m: (m, 0)),
            pl.BlockSpec((DIM, DIM), lambda m: (0, 0)),
            pl.BlockSpec((1, DIM), lambda m: (0, 0)),
        ],
        out_specs=pl.BlockSpec((BM, DIM), lambda m: (m, 0)),
        out_shape=jax.ShapeDtypeStruct((N_TOK, DIM), jnp.float32),
        compiler_params=pltpu.CompilerParams(
            allow_input_fusion=[False, True, False]
        ),
    )(x, wb, b2)
